# baseline (speedup 1.0000x reference)
import functools

import jax
import jax.numpy as jnp
from jax import lax
from jax.experimental import pallas as pl
from jax.experimental.pallas import tpu as pltpu
from jax.experimental.pallas import tpu_sc as plsc

N = 10000
E = 320000
D = 128

NC = 2
NS = 16
NW = NC * NS
CHUNK = 128
NCH = 80
EPW = NCH * CHUNK
E_PAD = EPW * NW
RPT = -(-(N + 1) // (NS * 8)) * 8
ACC_ROWS = RPT * NS
CW = 16


def _copy_out(c, s, acc_sh, agg_out):
    base = s * RPT
    last = N - (NS - 1) * RPT

    @pl.when(s < NS - 1)
    def _():
        pltpu.sync_copy(acc_sh.at[pl.ds(base, RPT)],
                        agg_out.at[c, pl.ds(base, RPT)])

    @pl.when(s == NS - 1)
    def _():
        pltpu.sync_copy(acc_sh.at[pl.ds(base, last)],
                        agg_out.at[c, pl.ds(base, last)])


def _sc_body_cnt(h, srcg, dstg, zacc, zcnt, agg_out, cnt_out,
                 src_v, dst_v, rows_v, cnt_priv, acc_sh, sem):
    c = lax.axis_index("c")
    s = lax.axis_index("s")
    gwid = c * NS + s

    pltpu.sync_copy(zacc, acc_sh.at[pl.ds(s * RPT, RPT)])
    pltpu.sync_copy(zcnt, cnt_priv)
    pltpu.sync_copy(srcg.at[gwid], src_v)
    pltpu.sync_copy(dstg.at[gwid], dst_v)
    plsc.subcore_barrier()

    ones16 = jnp.ones((16,), jnp.float32)

    def step(j, carry):
        pltpu.async_copy(h.at[src_v.at[j]], rows_v, sem).wait()
        pltpu.sync_copy(rows_v, acc_sh.at[dst_v.at[j]], add=True)
        for k in range(CHUNK // 16):
            idx = dst_v[j, pl.ds(k * 16, 16)]
            plsc.addupdate_scatter(cnt_priv, [idx], ones16)
        return carry

    lax.fori_loop(0, NCH, step, 0)
    plsc.subcore_barrier()
    _copy_out(c, s, acc_sh, agg_out)
    pltpu.sync_copy(cnt_priv, cnt_out.at[c, s])


def _sc_body_fast(h, srcg, dstg, zacc, agg_out,
                  src_v, dst_v, rows_v, acc_sh, sem):
    c = lax.axis_index("c")
    s = lax.axis_index("s")
    gwid = c * NS + s

    pltpu.sync_copy(zacc, acc_sh.at[pl.ds(s * RPT, RPT)])
    pltpu.sync_copy(srcg.at[gwid], src_v)
    pltpu.sync_copy(dstg.at[gwid], dst_v)
    plsc.subcore_barrier()

    def step(j, carry):
        pltpu.async_copy(h.at[src_v.at[j]], rows_v, sem).wait()
        pltpu.sync_copy(rows_v, acc_sh.at[dst_v.at[j]], add=True)
        return carry

    lax.fori_loop(0, NCH, step, 0)
    plsc.subcore_barrier()
    _copy_out(c, s, acc_sh, agg_out)


@functools.lru_cache(maxsize=None)
def _sc_kernels():
    mesh = plsc.VectorSubcoreMesh(core_axis_name="c", subcore_axis_name="s",
                                  num_cores=NC, num_subcores=NS)
    params = pltpu.CompilerParams(needs_layout_passes=False)
    agg_cnt = pl.kernel(
        _sc_body_cnt,
        out_type=(jax.ShapeDtypeStruct((NC, N, D), jnp.float32),
                  jax.ShapeDtypeStruct((NC, NS, ACC_ROWS), jnp.float32)),
        mesh=mesh,
        compiler_params=params,
        scratch_types=[
            pltpu.VMEM((NCH, CHUNK), jnp.int32),
            pltpu.VMEM((NCH, CHUNK), jnp.int32),
            pltpu.VMEM((CHUNK, D), jnp.float32),
            pltpu.VMEM((ACC_ROWS,), jnp.float32),
            pltpu.VMEM_SHARED((ACC_ROWS, D), jnp.float32),
            pltpu.SemaphoreType.DMA,
        ],
    )
    agg = pl.kernel(
        _sc_body_fast,
        out_type=jax.ShapeDtypeStruct((NC, N, D), jnp.float32),
        mesh=mesh,
        compiler_params=params,
        scratch_types=[
            pltpu.VMEM((NCH, CHUNK), jnp.int32),
            pltpu.VMEM((NCH, CHUNK), jnp.int32),
            pltpu.VMEM((CHUNK, D), jnp.float32),
            pltpu.VMEM_SHARED((ACC_ROWS, D), jnp.float32),
            pltpu.SemaphoreType.DMA,
        ],
    )
    return agg_cnt, agg


RB = 1000


def _tc_body(relu, p_ref, cnt_ref, h_ref, wlt_ref, wrt_ref, bl_ref, out_ref):
    cnt = jnp.sum(cnt_ref[...], axis=1, keepdims=True)
    rcp = 1.0 / jnp.maximum(cnt, 1.0)
    mean = (p_ref[0] + p_ref[1]) * rcp
    out = (jnp.dot(mean, wlt_ref[...], preferred_element_type=jnp.float32)
           + jnp.dot(h_ref[...], wrt_ref[...],
                     preferred_element_type=jnp.float32)
           + bl_ref[...])
    if relu:
        out = jnp.maximum(out, 0.0)
    out_ref[...] = out


def _tc_layer(p, cntp, h, wlt, wrt, bl, relu):
    grid = (N // RB,)
    return pl.pallas_call(
        functools.partial(_tc_body, relu),
        grid=grid,
        in_specs=[
            pl.BlockSpec((NC, RB, D), lambda i: (0, i, 0)),
            pl.BlockSpec((RB, NW), lambda i: (i, 0)),
            pl.BlockSpec((RB, D), lambda i: (i, 0)),
            pl.BlockSpec((D, D), lambda i: (0, 0)),
            pl.BlockSpec((D, D), lambda i: (0, 0)),
            pl.BlockSpec((1, D), lambda i: (0, 0)),
        ],
        out_specs=pl.BlockSpec((RB, D), lambda i: (i, 0)),
        out_shape=jax.ShapeDtypeStruct((N, D), jnp.float32),
    )(p, cntp, h, wlt, wrt, bl)


def kernel(x, edge_index, Wl0, bl0, Wr0, Wl1, bl1, Wr1, Wl2, bl2, Wr2):
    src = edge_index[0]
    dst = edge_index[1]
    pad = E_PAD - E
    src_p = jnp.concatenate([src, jnp.zeros((pad,), jnp.int32)])
    dst_p = jnp.concatenate([dst, jnp.full((pad,), N, jnp.int32)])
    srcg = src_p.reshape(NW, NCH, CHUNK)
    dstg = dst_p.reshape(NW, NCH, CHUNK)
    zacc = jnp.zeros((RPT, D), jnp.float32)
    zcnt = jnp.zeros((ACC_ROWS,), jnp.float32)

    sc_agg_cnt, sc_agg = _sc_kernels()
    a0, cntp = sc_agg_cnt(x, srcg, dstg, zacc, zcnt)
    cntp = cntp.reshape(NW, ACC_ROWS).T
    h1 = _tc_layer(a0, cntp, x, Wl0.T, Wr0.T, bl0.reshape(1, D), relu=True)
    a1 = sc_agg(h1, srcg, dstg, zacc)
    h2 = _tc_layer(a1, cntp, h1, Wl1.T, Wr1.T, bl1.reshape(1, D), relu=True)
    a2 = sc_agg(h2, srcg, dstg, zacc)
    h3 = _tc_layer(a2, cntp, h2, Wl2.T, Wr2.T, bl2.reshape(1, D), relu=False)
    return h3


# speedup vs baseline: 2.5725x; 2.5725x over previous
import functools

import jax
import jax.numpy as jnp
from jax import lax
from jax.experimental import pallas as pl
from jax.experimental.pallas import tpu as pltpu
from jax.experimental.pallas import tpu_sc as plsc

N = 10000
E = 320000
D = 128

NC = 2
NS = 16
NW = NC * NS
CHUNK = 128
NCH = -(-E // (NW * CHUNK))
EPW = NCH * CHUNK
E_PAD = EPW * NW
RPT = -(-(N + 1) // (NS * 8)) * 8
ACC_ROWS = RPT * NS
CW = 16


def _copy_out(c, s, acc_sh, agg_out):
    base = s * RPT
    last = N - (NS - 1) * RPT

    @pl.when(s < NS - 1)
    def _():
        pltpu.sync_copy(acc_sh.at[pl.ds(base, RPT)],
                        agg_out.at[c, pl.ds(base, RPT)])

    @pl.when(s == NS - 1)
    def _():
        pltpu.sync_copy(acc_sh.at[pl.ds(base, last)],
                        agg_out.at[c, pl.ds(base, last)])


def _sc_body_cnt(h, srcg, dstg, zacc, zcnt, agg_out, cnt_out,
                 src_v, dst_v, rows_v, cnt_priv, acc_sh, sem):
    c = lax.axis_index("c")
    s = lax.axis_index("s")
    gwid = c * NS + s

    pltpu.sync_copy(zacc, acc_sh.at[pl.ds(s * RPT, RPT)])
    pltpu.sync_copy(zcnt, cnt_priv)
    pltpu.sync_copy(srcg.at[gwid], src_v)
    pltpu.sync_copy(dstg.at[gwid], dst_v)
    plsc.subcore_barrier()

    ones16 = jnp.ones((16,), jnp.float32)

    def step(j, carry):
        pltpu.async_copy(h.at[src_v.at[j]], rows_v, sem).wait()
        pltpu.sync_copy(rows_v, acc_sh.at[dst_v.at[j]], add=True)
        for k in range(CHUNK // 16):
            idx = dst_v[j, pl.ds(k * 16, 16)]
            plsc.addupdate_scatter(cnt_priv, [idx], ones16)
        return carry

    lax.fori_loop(0, NCH, step, 0)
    plsc.subcore_barrier()
    _copy_out(c, s, acc_sh, agg_out)
    pltpu.sync_copy(cnt_priv, cnt_out.at[c, s])


def _sc_body_fast(h, srcg, dstg, zacc, agg_out,
                  src_v, dst_v, rows_v, acc_sh, sem):
    c = lax.axis_index("c")
    s = lax.axis_index("s")
    gwid = c * NS + s

    pltpu.sync_copy(zacc, acc_sh.at[pl.ds(s * RPT, RPT)])
    pltpu.sync_copy(srcg.at[gwid], src_v)
    pltpu.sync_copy(dstg.at[gwid], dst_v)
    plsc.subcore_barrier()

    def step(j, carry):
        pltpu.async_copy(h.at[src_v.at[j]], rows_v, sem).wait()
        pltpu.sync_copy(rows_v, acc_sh.at[dst_v.at[j]], add=True)
        return carry

    lax.fori_loop(0, NCH, step, 0)
    plsc.subcore_barrier()
    _copy_out(c, s, acc_sh, agg_out)


@functools.lru_cache(maxsize=None)
def _sc_kernels():
    mesh = plsc.VectorSubcoreMesh(core_axis_name="c", subcore_axis_name="s",
                                  num_cores=NC, num_subcores=NS)
    params = pltpu.CompilerParams(needs_layout_passes=False)
    agg_cnt = pl.kernel(
        _sc_body_cnt,
        out_type=(jax.ShapeDtypeStruct((NC, N, D), jnp.float32),
                  jax.ShapeDtypeStruct((NC, NS, ACC_ROWS), jnp.float32)),
        mesh=mesh,
        compiler_params=params,
        scratch_types=[
            pltpu.VMEM((NCH, CHUNK), jnp.int32),
            pltpu.VMEM((NCH, CHUNK), jnp.int32),
            pltpu.VMEM((CHUNK, D), jnp.float32),
            pltpu.VMEM((ACC_ROWS,), jnp.float32),
            pltpu.VMEM_SHARED((ACC_ROWS, D), jnp.float32),
            pltpu.SemaphoreType.DMA,
        ],
    )
    agg = pl.kernel(
        _sc_body_fast,
        out_type=jax.ShapeDtypeStruct((NC, N, D), jnp.float32),
        mesh=mesh,
        compiler_params=params,
        scratch_types=[
            pltpu.VMEM((NCH, CHUNK), jnp.int32),
            pltpu.VMEM((NCH, CHUNK), jnp.int32),
            pltpu.VMEM((CHUNK, D), jnp.float32),
            pltpu.VMEM_SHARED((ACC_ROWS, D), jnp.float32),
            pltpu.SemaphoreType.DMA,
        ],
    )
    return agg_cnt, agg


RB = 1000


def _tc_body(relu, p_ref, cnt_ref, h_ref, wlt_ref, wrt_ref, bl_ref, out_ref):
    cnt = jnp.sum(cnt_ref[...], axis=1, keepdims=True)
    rcp = 1.0 / jnp.maximum(cnt, 1.0)
    mean = (p_ref[0] + p_ref[1]) * rcp
    out = (jnp.dot(mean, wlt_ref[...], preferred_element_type=jnp.float32)
           + jnp.dot(h_ref[...], wrt_ref[...],
                     preferred_element_type=jnp.float32)
           + bl_ref[...])
    if relu:
        out = jnp.maximum(out, 0.0)
    out_ref[...] = out


def _tc_layer(p, cntp, h, wlt, wrt, bl, relu):
    grid = (N // RB,)
    return pl.pallas_call(
        functools.partial(_tc_body, relu),
        grid=grid,
        in_specs=[
            pl.BlockSpec((NC, RB, D), lambda i: (0, i, 0)),
            pl.BlockSpec((RB, NW), lambda i: (i, 0)),
            pl.BlockSpec((RB, D), lambda i: (i, 0)),
            pl.BlockSpec((D, D), lambda i: (0, 0)),
            pl.BlockSpec((D, D), lambda i: (0, 0)),
            pl.BlockSpec((1, D), lambda i: (0, 0)),
        ],
        out_specs=pl.BlockSpec((RB, D), lambda i: (i, 0)),
        out_shape=jax.ShapeDtypeStruct((N, D), jnp.float32),
    )(p, cntp, h, wlt, wrt, bl)


def kernel(x, edge_index, Wl0, bl0, Wr0, Wl1, bl1, Wr1, Wl2, bl2, Wr2):
    src = edge_index[0]
    dst = edge_index[1]
    pad = E_PAD - E
    pad_src = (jnp.arange(pad, dtype=jnp.int32) * 31) % N
    pad_dst = N + (jnp.arange(pad, dtype=jnp.int32) % (ACC_ROWS - N))
    src_p = jnp.concatenate([src, pad_src])
    dst_p = jnp.concatenate([dst, pad_dst])
    srcg = src_p.reshape(EPW, NW).T.reshape(NW, NCH, CHUNK)
    dstg = dst_p.reshape(EPW, NW).T.reshape(NW, NCH, CHUNK)
    zacc = jnp.zeros((RPT, D), jnp.float32)
    zcnt = jnp.zeros((ACC_ROWS,), jnp.float32)

    sc_agg_cnt, sc_agg = _sc_kernels()
    a0, cntp = sc_agg_cnt(x, srcg, dstg, zacc, zcnt)
    cntp = cntp.reshape(NW, ACC_ROWS).T
    h1 = _tc_layer(a0, cntp, x, Wl0.T, Wr0.T, bl0.reshape(1, D), relu=True)
    a1 = sc_agg(h1, srcg, dstg, zacc)
    h2 = _tc_layer(a1, cntp, h1, Wl1.T, Wr1.T, bl1.reshape(1, D), relu=True)
    a2 = sc_agg(h2, srcg, dstg, zacc)
    h3 = _tc_layer(a2, cntp, h2, Wl2.T, Wr2.T, bl2.reshape(1, D), relu=False)
    return h3


# speedup vs baseline: 2.8309x; 1.1005x over previous
import functools

import jax
import jax.numpy as jnp
from jax import lax
from jax.experimental import pallas as pl
from jax.experimental.pallas import tpu as pltpu
from jax.experimental.pallas import tpu_sc as plsc

N = 10000
E = 320000
D = 128

NC = 2
NS = 16
NW = NC * NS
CHUNK = 128
NCH = 80
EPW = NCH * CHUNK
E_PAD = EPW * NW
RPT = -(-(N + 1) // (NS * 8)) * 8
ACC_ROWS = RPT * NS
CW = 16


def _copy_out(c, s, acc_sh, agg_out):
    base = s * RPT
    last = N - (NS - 1) * RPT

    @pl.when(s < NS - 1)
    def _():
        pltpu.sync_copy(acc_sh.at[pl.ds(base, RPT)],
                        agg_out.at[c, pl.ds(base, RPT)])

    @pl.when(s == NS - 1)
    def _():
        pltpu.sync_copy(acc_sh.at[pl.ds(base, last)],
                        agg_out.at[c, pl.ds(base, last)])


def _sc_body_cnt(h, srcg, dstg, zacc, zcnt, agg_out, cnt_out,
                 src_v, dst_v, rows_v, cnt_priv, acc_sh, sem):
    c = lax.axis_index("c")
    s = lax.axis_index("s")
    gwid = c * NS + s

    pltpu.sync_copy(zacc, acc_sh.at[pl.ds(s * RPT, RPT)])
    pltpu.sync_copy(zcnt, cnt_priv)
    pltpu.sync_copy(srcg.at[gwid], src_v)
    pltpu.sync_copy(dstg.at[gwid], dst_v)
    plsc.subcore_barrier()

    ones16 = jnp.ones((16,), jnp.float32)

    def step(j, carry):
        pltpu.async_copy(h.at[src_v.at[j]], rows_v, sem).wait()
        pltpu.sync_copy(rows_v, acc_sh.at[dst_v.at[j]], add=True)
        for k in range(CHUNK // 16):
            idx = dst_v[j, pl.ds(k * 16, 16)]
            plsc.addupdate_scatter(cnt_priv, [idx], ones16)
        return carry

    lax.fori_loop(0, NCH, step, 0)
    plsc.subcore_barrier()
    _copy_out(c, s, acc_sh, agg_out)
    pltpu.sync_copy(cnt_priv, cnt_out.at[c, s])


def _sc_body_fast(h, srcg, dstg, zacc, agg_out,
                  src_i, dst_v, rows_a, rows_b, acc_sh,
                  sem_a, sem_b, sem_sa, sem_sb, sem_ia, sem_ib):
    c = lax.axis_index("c")
    s = lax.axis_index("s")
    gwid = c * NS + s

    pltpu.sync_copy(zacc, acc_sh.at[pl.ds(s * RPT, RPT)])
    pltpu.sync_copy(dstg.at[gwid], dst_v)
    plsc.subcore_barrier()

    def fire_idx(j, p, sem_i):
        pltpu.async_copy(srcg.at[gwid, j], src_i.at[p], sem_i)

    def wait_idx(p, sem_i):
        pltpu.make_async_copy(srcg.at[gwid, 0], src_i.at[p], sem_i).wait()

    def fire_g(p, buf, sem):
        pltpu.async_copy(h.at[src_i.at[p]], buf, sem)

    def drain_g(buf, sem):
        pltpu.make_async_copy(h.at[src_i.at[0]], buf, sem).wait()

    def fire_s(j, buf, sem):
        pltpu.async_copy(buf, acc_sh.at[dst_v.at[j]], sem, add=True)

    def drain_s(buf, sem):
        pltpu.make_async_copy(buf, acc_sh.at[dst_v.at[0]], sem).wait()

    fire_idx(0, 0, sem_ia)
    fire_idx(1, 1, sem_ib)
    wait_idx(0, sem_ia)
    fire_g(0, rows_a, sem_a)
    wait_idx(1, sem_ib)
    fire_g(1, rows_b, sem_b)

    def step(i, carry):
        j0 = 2 * i
        drain_g(rows_a, sem_a)
        fire_s(j0, rows_a, sem_sa)
        fire_idx(j0 + 2, 0, sem_ia)
        drain_g(rows_b, sem_b)
        fire_s(j0 + 1, rows_b, sem_sb)
        fire_idx(j0 + 3, 1, sem_ib)
        drain_s(rows_a, sem_sa)
        wait_idx(0, sem_ia)
        fire_g(0, rows_a, sem_a)
        drain_s(rows_b, sem_sb)
        wait_idx(1, sem_ib)
        fire_g(1, rows_b, sem_b)
        return carry

    lax.fori_loop(0, NCH // 2 - 1, step, 0)
    drain_g(rows_a, sem_a)
    fire_s(NCH - 2, rows_a, sem_sa)
    drain_g(rows_b, sem_b)
    fire_s(NCH - 1, rows_b, sem_sb)
    drain_s(rows_a, sem_sa)
    drain_s(rows_b, sem_sb)
    plsc.subcore_barrier()
    _copy_out(c, s, acc_sh, agg_out)


@functools.lru_cache(maxsize=None)
def _sc_kernels():
    mesh = plsc.VectorSubcoreMesh(core_axis_name="c", subcore_axis_name="s",
                                  num_cores=NC, num_subcores=NS)
    params = pltpu.CompilerParams(needs_layout_passes=False)
    agg_cnt = pl.kernel(
        _sc_body_cnt,
        out_type=(jax.ShapeDtypeStruct((NC, N, D), jnp.float32),
                  jax.ShapeDtypeStruct((NC, NS, ACC_ROWS), jnp.float32)),
        mesh=mesh,
        compiler_params=params,
        scratch_types=[
            pltpu.VMEM((NCH, CHUNK), jnp.int32),
            pltpu.VMEM((NCH, CHUNK), jnp.int32),
            pltpu.VMEM((CHUNK, D), jnp.float32),
            pltpu.VMEM((ACC_ROWS,), jnp.float32),
            pltpu.VMEM_SHARED((ACC_ROWS, D), jnp.float32),
            pltpu.SemaphoreType.DMA,
        ],
    )
    agg = pl.kernel(
        _sc_body_fast,
        out_type=jax.ShapeDtypeStruct((NC, N, D), jnp.float32),
        mesh=mesh,
        compiler_params=params,
        scratch_types=[
            pltpu.VMEM((2, CHUNK), jnp.int32),
            pltpu.VMEM((NCH, CHUNK), jnp.int32),
            pltpu.VMEM((CHUNK, D), jnp.float32),
            pltpu.VMEM((CHUNK, D), jnp.float32),
            pltpu.VMEM_SHARED((ACC_ROWS, D), jnp.float32),
            pltpu.SemaphoreType.DMA,
            pltpu.SemaphoreType.DMA,
            pltpu.SemaphoreType.DMA,
            pltpu.SemaphoreType.DMA,
            pltpu.SemaphoreType.DMA,
            pltpu.SemaphoreType.DMA,
        ],
    )
    return agg_cnt, agg


RB = 1000


def _tc_body(relu, p_ref, cnt_ref, h_ref, wlt_ref, wrt_ref, bl_ref, out_ref):
    cnt = jnp.sum(cnt_ref[...], axis=1, keepdims=True)
    rcp = 1.0 / jnp.maximum(cnt, 1.0)
    mean = (p_ref[0] + p_ref[1]) * rcp
    out = (jnp.dot(mean, wlt_ref[...], preferred_element_type=jnp.float32)
           + jnp.dot(h_ref[...], wrt_ref[...],
                     preferred_element_type=jnp.float32)
           + bl_ref[...])
    if relu:
        out = jnp.maximum(out, 0.0)
    out_ref[...] = out


def _tc_layer(p, cntp, h, wlt, wrt, bl, relu):
    grid = (N // RB,)
    return pl.pallas_call(
        functools.partial(_tc_body, relu),
        grid=grid,
        in_specs=[
            pl.BlockSpec((NC, RB, D), lambda i: (0, i, 0)),
            pl.BlockSpec((RB, NW), lambda i: (i, 0)),
            pl.BlockSpec((RB, D), lambda i: (i, 0)),
            pl.BlockSpec((D, D), lambda i: (0, 0)),
            pl.BlockSpec((D, D), lambda i: (0, 0)),
            pl.BlockSpec((1, D), lambda i: (0, 0)),
        ],
        out_specs=pl.BlockSpec((RB, D), lambda i: (i, 0)),
        out_shape=jax.ShapeDtypeStruct((N, D), jnp.float32),
    )(p, cntp, h, wlt, wrt, bl)


def kernel(x, edge_index, Wl0, bl0, Wr0, Wl1, bl1, Wr1, Wl2, bl2, Wr2):
    src = edge_index[0]
    dst = edge_index[1]
    pad = E_PAD - E
    pad_src = (jnp.arange(pad, dtype=jnp.int32) * 31) % N
    pad_dst = N + (jnp.arange(pad, dtype=jnp.int32) % (ACC_ROWS - N))
    src_p = jnp.concatenate([src, pad_src])
    dst_p = jnp.concatenate([dst, pad_dst])
    srcg = src_p.reshape(EPW, NW).T.reshape(NW, NCH, CHUNK)
    dstg = dst_p.reshape(EPW, NW).T.reshape(NW, NCH, CHUNK)
    zacc = jnp.zeros((RPT, D), jnp.float32)
    zcnt = jnp.zeros((ACC_ROWS,), jnp.float32)

    sc_agg_cnt, sc_agg = _sc_kernels()
    a0, cntp = sc_agg_cnt(x, srcg, dstg, zacc, zcnt)
    cntp = cntp.reshape(NW, ACC_ROWS).T
    h1 = _tc_layer(a0, cntp, x, Wl0.T, Wr0.T, bl0.reshape(1, D), relu=True)
    a1 = sc_agg(h1, srcg, dstg, zacc)
    h2 = _tc_layer(a1, cntp, h1, Wl1.T, Wr1.T, bl1.reshape(1, D), relu=True)
    a2 = sc_agg(h2, srcg, dstg, zacc)
    h3 = _tc_layer(a2, cntp, h2, Wl2.T, Wr2.T, bl2.reshape(1, D), relu=False)
    return h3


# speedup vs baseline: 3.0889x; 1.0911x over previous
import functools

import jax
import jax.numpy as jnp
from jax import lax
from jax.experimental import pallas as pl
from jax.experimental.pallas import tpu as pltpu
from jax.experimental.pallas import tpu_sc as plsc

N = 10000
E = 320000
D = 128

NC = 2
NS = 16
NW = NC * NS
CHUNK = 128
NCH = 80
EPW = NCH * CHUNK
E_PAD = EPW * NW
RPT = -(-(N + 1) // (NS * 8)) * 8
ACC_ROWS = RPT * NS
CW = 16


def _copy_out(c, s, acc_sh, agg_out):
    base = s * RPT
    last = N - (NS - 1) * RPT

    @pl.when(s < NS - 1)
    def _():
        pltpu.sync_copy(acc_sh.at[pl.ds(base, RPT)],
                        agg_out.at[c, pl.ds(base, RPT)])

    @pl.when(s == NS - 1)
    def _():
        pltpu.sync_copy(acc_sh.at[pl.ds(base, last)],
                        agg_out.at[c, pl.ds(base, last)])


def _zero_acc(s, rows_a, acc_sh):
    z16 = jnp.zeros((16,), jnp.float32)

    def zrow(r, carry):
        for k in range(CHUNK // 16):
            rows_a[r, pl.ds(k * 16, 16)] = z16
        return carry

    lax.fori_loop(0, CHUNK, zrow, 0)
    base = s * RPT
    for t in range(RPT // CHUNK):
        pltpu.sync_copy(rows_a, acc_sh.at[pl.ds(base + t * CHUNK, CHUNK)])
    rem = RPT % CHUNK
    pltpu.sync_copy(rows_a.at[pl.ds(0, rem)],
                    acc_sh.at[pl.ds(base + RPT - rem, rem)])


def _ring(h, srcg, gwid, base_j, nch, src_i, dst_v, rows_a, rows_b, acc_sh,
          sem_a, sem_b, sem_sa, sem_sb, sem_ia, sem_ib, cnt_priv=None):
    ones16 = jnp.ones((16,), jnp.float32)

    def fire_idx(j, p, sem_i):
        pltpu.async_copy(srcg.at[gwid, base_j + j], src_i.at[p], sem_i)

    def wait_idx(p, sem_i):
        pltpu.make_async_copy(srcg.at[gwid, 0], src_i.at[p], sem_i).wait()

    def fire_g(p, buf, sem):
        pltpu.async_copy(h.at[src_i.at[p]], buf, sem)

    def drain_g(buf, sem):
        pltpu.make_async_copy(h.at[src_i.at[0]], buf, sem).wait()

    def fire_s(j, buf, sem):
        pltpu.async_copy(buf, acc_sh.at[dst_v.at[j]], sem, add=True)
        if cnt_priv is not None:
            for k in range(CHUNK // 16):
                idx = dst_v[j, pl.ds(k * 16, 16)]
                plsc.addupdate_scatter(cnt_priv, [idx], ones16)

    def drain_s(buf, sem):
        pltpu.make_async_copy(buf, acc_sh.at[dst_v.at[0]], sem).wait()

    fire_idx(0, 0, sem_ia)
    fire_idx(1, 1, sem_ib)
    wait_idx(0, sem_ia)
    fire_g(0, rows_a, sem_a)
    wait_idx(1, sem_ib)
    fire_g(1, rows_b, sem_b)

    def step(i, carry):
        j0 = 2 * i
        drain_g(rows_a, sem_a)
        fire_s(j0, rows_a, sem_sa)
        fire_idx(j0 + 2, 0, sem_ia)
        drain_g(rows_b, sem_b)
        fire_s(j0 + 1, rows_b, sem_sb)
        fire_idx(j0 + 3, 1, sem_ib)
        drain_s(rows_a, sem_sa)
        wait_idx(0, sem_ia)
        fire_g(0, rows_a, sem_a)
        drain_s(rows_b, sem_sb)
        wait_idx(1, sem_ib)
        fire_g(1, rows_b, sem_b)
        return carry

    lax.fori_loop(0, nch // 2 - 1, step, 0)
    drain_g(rows_a, sem_a)
    fire_s(nch - 2, rows_a, sem_sa)
    drain_g(rows_b, sem_b)
    fire_s(nch - 1, rows_b, sem_sb)
    drain_s(rows_a, sem_sa)
    drain_s(rows_b, sem_sb)


NHALF = NCH // 2


def _sc_body_cnt(h, srcg, dstg, agg_out, cnt_out,
                 src_i, dst_v, rows_a, rows_b, cnt_priv, acc_sh,
                 sem_a, sem_b, sem_sa, sem_sb, sem_ia, sem_ib):
    c = lax.axis_index("c")
    s = lax.axis_index("s")
    gwid = c * NS + s

    _zero_acc(s, rows_a, acc_sh)
    z16 = jnp.zeros((16,), jnp.float32)

    def zcnt(i, carry):
        cnt_priv[pl.ds(i * 16, 16)] = z16
        return carry

    lax.fori_loop(0, ACC_ROWS // 16, zcnt, 0)
    plsc.subcore_barrier()

    for hb in range(2):
        pltpu.sync_copy(dstg.at[gwid, pl.ds(hb * NHALF, NHALF)], dst_v)
        _ring(h, srcg, gwid, hb * NHALF, NHALF, src_i, dst_v,
              rows_a, rows_b, acc_sh,
              sem_a, sem_b, sem_sa, sem_sb, sem_ia, sem_ib,
              cnt_priv=cnt_priv)
    plsc.subcore_barrier()
    _copy_out(c, s, acc_sh, agg_out)
    pltpu.sync_copy(cnt_priv, cnt_out.at[c, s])


def _sc_body_fast(h, srcg, dstg, agg_out,
                  src_i, dst_v, rows_a, rows_b, acc_sh,
                  sem_a, sem_b, sem_sa, sem_sb, sem_ia, sem_ib):
    c = lax.axis_index("c")
    s = lax.axis_index("s")
    gwid = c * NS + s

    _zero_acc(s, rows_a, acc_sh)
    pltpu.sync_copy(dstg.at[gwid], dst_v)
    plsc.subcore_barrier()

    _ring(h, srcg, gwid, 0, NCH, src_i, dst_v, rows_a, rows_b, acc_sh,
          sem_a, sem_b, sem_sa, sem_sb, sem_ia, sem_ib)
    plsc.subcore_barrier()
    _copy_out(c, s, acc_sh, agg_out)


@functools.lru_cache(maxsize=None)
def _sc_kernels():
    mesh = plsc.VectorSubcoreMesh(core_axis_name="c", subcore_axis_name="s",
                                  num_cores=NC, num_subcores=NS)
    params = pltpu.CompilerParams(needs_layout_passes=False)
    agg_cnt = pl.kernel(
        _sc_body_cnt,
        out_type=(jax.ShapeDtypeStruct((NC, N, D), jnp.float32),
                  jax.ShapeDtypeStruct((NC, NS, ACC_ROWS), jnp.float32)),
        mesh=mesh,
        compiler_params=params,
        scratch_types=[
            pltpu.VMEM((2, CHUNK), jnp.int32),
            pltpu.VMEM((NHALF, CHUNK), jnp.int32),
            pltpu.VMEM((CHUNK, D), jnp.float32),
            pltpu.VMEM((CHUNK, D), jnp.float32),
            pltpu.VMEM((ACC_ROWS,), jnp.float32),
            pltpu.VMEM_SHARED((ACC_ROWS, D), jnp.float32),
        ] + [pltpu.SemaphoreType.DMA] * 6,
    )
    agg = pl.kernel(
        _sc_body_fast,
        out_type=jax.ShapeDtypeStruct((NC, N, D), jnp.float32),
        mesh=mesh,
        compiler_params=params,
        scratch_types=[
            pltpu.VMEM((2, CHUNK), jnp.int32),
            pltpu.VMEM((NCH, CHUNK), jnp.int32),
            pltpu.VMEM((CHUNK, D), jnp.float32),
            pltpu.VMEM((CHUNK, D), jnp.float32),
            pltpu.VMEM_SHARED((ACC_ROWS, D), jnp.float32),
        ] + [pltpu.SemaphoreType.DMA] * 6,
    )
    return agg_cnt, agg


RB = 1000


def _tc_body(relu, p_ref, cnt_ref, h_ref, wlt_ref, wrt_ref, bl_ref, out_ref):
    cnt = jnp.sum(cnt_ref[...], axis=1, keepdims=True)
    rcp = 1.0 / jnp.maximum(cnt, 1.0)
    mean = (p_ref[0] + p_ref[1]) * rcp
    out = (jnp.dot(mean, wlt_ref[...], preferred_element_type=jnp.float32)
           + jnp.dot(h_ref[...], wrt_ref[...],
                     preferred_element_type=jnp.float32)
           + bl_ref[...])
    if relu:
        out = jnp.maximum(out, 0.0)
    out_ref[...] = out


def _tc_layer(p, cntp, h, wlt, wrt, bl, relu):
    grid = (N // RB,)
    return pl.pallas_call(
        functools.partial(_tc_body, relu),
        grid=grid,
        in_specs=[
            pl.BlockSpec((NC, RB, D), lambda i: (0, i, 0)),
            pl.BlockSpec((RB, NW), lambda i: (i, 0)),
            pl.BlockSpec((RB, D), lambda i: (i, 0)),
            pl.BlockSpec((D, D), lambda i: (0, 0)),
            pl.BlockSpec((D, D), lambda i: (0, 0)),
            pl.BlockSpec((1, D), lambda i: (0, 0)),
        ],
        out_specs=pl.BlockSpec((RB, D), lambda i: (i, 0)),
        out_shape=jax.ShapeDtypeStruct((N, D), jnp.float32),
    )(p, cntp, h, wlt, wrt, bl)


def kernel(x, edge_index, Wl0, bl0, Wr0, Wl1, bl1, Wr1, Wl2, bl2, Wr2):
    src = edge_index[0]
    dst = edge_index[1]
    pad = E_PAD - E
    pad_src = (jnp.arange(pad, dtype=jnp.int32) * 31) % N
    pad_dst = N + (jnp.arange(pad, dtype=jnp.int32) % (ACC_ROWS - N))
    src_p = jnp.concatenate([src, pad_src])
    dst_p = jnp.concatenate([dst, pad_dst])
    srcg = src_p.reshape(EPW, NW).T.reshape(NW, NCH, CHUNK)
    dstg = dst_p.reshape(EPW, NW).T.reshape(NW, NCH, CHUNK)
    sc_agg_cnt, sc_agg = _sc_kernels()
    a0, cntp = sc_agg_cnt(x, srcg, dstg)
    cntp = cntp.reshape(NW, ACC_ROWS).T
    h1 = _tc_layer(a0, cntp, x, Wl0.T, Wr0.T, bl0.reshape(1, D), relu=True)
    a1 = sc_agg(h1, srcg, dstg)
    h2 = _tc_layer(a1, cntp, h1, Wl1.T, Wr1.T, bl1.reshape(1, D), relu=True)
    a2 = sc_agg(h2, srcg, dstg)
    h3 = _tc_layer(a2, cntp, h2, Wl2.T, Wr2.T, bl2.reshape(1, D), relu=False)
    return h3


# speedup vs baseline: 3.3149x; 1.0732x over previous
import functools

import jax
import jax.numpy as jnp
from jax import lax
from jax.experimental import pallas as pl
from jax.experimental.pallas import tpu as pltpu
from jax.experimental.pallas import tpu_sc as plsc

N = 10000
E = 320000
D = 128

NC = 2
NS = 16
NW = NC * NS
CHUNK = 128
NCH = 80
EPW = NCH * CHUNK
E_PAD = EPW * NW
RPT = -(-(N + 1) // (NS * 8)) * 8
ACC_ROWS = RPT * NS
CW = 16


def _copy_out(c, s, acc_sh, agg_out):
    base = s * RPT
    last = N - (NS - 1) * RPT

    @pl.when(s < NS - 1)
    def _():
        pltpu.sync_copy(acc_sh.at[pl.ds(base, RPT)],
                        agg_out.at[c, pl.ds(base, RPT)])

    @pl.when(s == NS - 1)
    def _():
        pltpu.sync_copy(acc_sh.at[pl.ds(base, last)],
                        agg_out.at[c, pl.ds(base, last)])


def _zero_acc(s, rows_a, acc_sh):
    z16 = jnp.zeros((16,), jnp.float32)

    def zrow(r, carry):
        for k in range(CHUNK // 16):
            rows_a[r, pl.ds(k * 16, 16)] = z16
        return carry

    lax.fori_loop(0, CHUNK, zrow, 0)
    base = s * RPT
    for t in range(RPT // CHUNK):
        pltpu.sync_copy(rows_a, acc_sh.at[pl.ds(base + t * CHUNK, CHUNK)])
    rem = RPT % CHUNK
    pltpu.sync_copy(rows_a.at[pl.ds(0, rem)],
                    acc_sh.at[pl.ds(base + RPT - rem, rem)])


def _ring(h, srcg, gwid, base_j, nch, src_i, dst_v, rows_a, rows_b, acc_sh,
          sem_a, sem_b, sem_sa, sem_sb, sem_ia, sem_ib, cnt_priv=None):
    ones16 = jnp.ones((16,), jnp.float32)

    def fire_idx(j, p, sem_i):
        pltpu.async_copy(srcg.at[gwid, base_j + j], src_i.at[p], sem_i)

    def wait_idx(p, sem_i):
        pltpu.make_async_copy(srcg.at[gwid, 0], src_i.at[p], sem_i).wait()

    def fire_g(p, buf, sem):
        pltpu.async_copy(h.at[src_i.at[p]], buf, sem)

    def drain_g(buf, sem):
        pltpu.make_async_copy(h.at[src_i.at[0]], buf, sem).wait()

    def fire_s(j, buf, sem):
        pltpu.async_copy(buf, acc_sh.at[dst_v.at[j]], sem, add=True)
        if cnt_priv is not None:
            for k in range(CHUNK // 16):
                idx = dst_v[j, pl.ds(k * 16, 16)]
                plsc.addupdate_scatter(cnt_priv, [idx], ones16)

    def drain_s(buf, sem):
        pltpu.make_async_copy(buf, acc_sh.at[dst_v.at[0]], sem).wait()

    fire_idx(0, 0, sem_ia)
    fire_idx(1, 1, sem_ib)
    wait_idx(0, sem_ia)
    fire_g(0, rows_a, sem_a)
    drain_g(rows_a, sem_a)
    fire_s(0, rows_a, sem_sa)
    fire_idx(2, 0, sem_ia)
    wait_idx(1, sem_ib)
    fire_g(1, rows_b, sem_b)

    def step(i, carry):
        j0 = 2 * i
        drain_g(rows_b, sem_b)
        fire_s(j0 + 1, rows_b, sem_sb)
        fire_idx(j0 + 3, 1, sem_ib)
        drain_s(rows_a, sem_sa)
        wait_idx(0, sem_ia)
        fire_g(0, rows_a, sem_a)
        drain_g(rows_a, sem_a)
        fire_s(j0 + 2, rows_a, sem_sa)
        fire_idx(j0 + 4, 0, sem_ia)
        drain_s(rows_b, sem_sb)
        wait_idx(1, sem_ib)
        fire_g(1, rows_b, sem_b)
        return carry

    lax.fori_loop(0, nch // 2 - 1, step, 0)
    drain_g(rows_b, sem_b)
    fire_s(nch - 1, rows_b, sem_sb)
    drain_s(rows_a, sem_sa)
    drain_s(rows_b, sem_sb)
    wait_idx(0, sem_ia)


NHALF = NCH // 2


def _sc_body_cnt(h, srcg, dstg, agg_out, cnt_out,
                 src_i, dst_v, rows_a, rows_b, cnt_priv, acc_sh,
                 sem_a, sem_b, sem_sa, sem_sb, sem_ia, sem_ib):
    c = lax.axis_index("c")
    s = lax.axis_index("s")
    gwid = c * NS + s

    _zero_acc(s, rows_a, acc_sh)
    z16 = jnp.zeros((16,), jnp.float32)

    def zcnt(i, carry):
        cnt_priv[pl.ds(i * 16, 16)] = z16
        return carry

    lax.fori_loop(0, ACC_ROWS // 16, zcnt, 0)
    plsc.subcore_barrier()

    for hb in range(2):
        pltpu.sync_copy(dstg.at[gwid, pl.ds(hb * NHALF, NHALF)], dst_v)
        _ring(h, srcg, gwid, hb * NHALF, NHALF, src_i, dst_v,
              rows_a, rows_b, acc_sh,
              sem_a, sem_b, sem_sa, sem_sb, sem_ia, sem_ib,
              cnt_priv=cnt_priv)
    plsc.subcore_barrier()
    _copy_out(c, s, acc_sh, agg_out)
    pltpu.sync_copy(cnt_priv, cnt_out.at[c, s])


def _sc_body_fast(h, srcg, dstg, agg_out,
                  src_i, dst_v, rows_a, rows_b, acc_sh,
                  sem_a, sem_b, sem_sa, sem_sb, sem_ia, sem_ib):
    c = lax.axis_index("c")
    s = lax.axis_index("s")
    gwid = c * NS + s

    _zero_acc(s, rows_a, acc_sh)
    pltpu.sync_copy(dstg.at[gwid], dst_v)
    plsc.subcore_barrier()

    _ring(h, srcg, gwid, 0, NCH, src_i, dst_v, rows_a, rows_b, acc_sh,
          sem_a, sem_b, sem_sa, sem_sb, sem_ia, sem_ib)
    plsc.subcore_barrier()
    _copy_out(c, s, acc_sh, agg_out)


@functools.lru_cache(maxsize=None)
def _sc_kernels():
    mesh = plsc.VectorSubcoreMesh(core_axis_name="c", subcore_axis_name="s",
                                  num_cores=NC, num_subcores=NS)
    params = pltpu.CompilerParams(needs_layout_passes=False)
    agg_cnt = pl.kernel(
        _sc_body_cnt,
        out_type=(jax.ShapeDtypeStruct((NC, N, D), jnp.float32),
                  jax.ShapeDtypeStruct((NC, NS, ACC_ROWS), jnp.float32)),
        mesh=mesh,
        compiler_params=params,
        scratch_types=[
            pltpu.VMEM((2, CHUNK), jnp.int32),
            pltpu.VMEM((NHALF, CHUNK), jnp.int32),
            pltpu.VMEM((CHUNK, D), jnp.float32),
            pltpu.VMEM((CHUNK, D), jnp.float32),
            pltpu.VMEM((ACC_ROWS,), jnp.float32),
            pltpu.VMEM_SHARED((ACC_ROWS, D), jnp.float32),
        ] + [pltpu.SemaphoreType.DMA] * 6,
    )
    agg = pl.kernel(
        _sc_body_fast,
        out_type=jax.ShapeDtypeStruct((NC, N, D), jnp.float32),
        mesh=mesh,
        compiler_params=params,
        scratch_types=[
            pltpu.VMEM((2, CHUNK), jnp.int32),
            pltpu.VMEM((NCH, CHUNK), jnp.int32),
            pltpu.VMEM((CHUNK, D), jnp.float32),
            pltpu.VMEM((CHUNK, D), jnp.float32),
            pltpu.VMEM_SHARED((ACC_ROWS, D), jnp.float32),
        ] + [pltpu.SemaphoreType.DMA] * 6,
    )
    return agg_cnt, agg


RB = 1000


def _tc_body(relu, p_ref, cnt_ref, h_ref, wlt_ref, wrt_ref, bl_ref, out_ref):
    cnt = jnp.sum(cnt_ref[...], axis=1, keepdims=True)
    rcp = 1.0 / jnp.maximum(cnt, 1.0)
    mean = (p_ref[0] + p_ref[1]) * rcp
    out = (jnp.dot(mean, wlt_ref[...], preferred_element_type=jnp.float32)
           + jnp.dot(h_ref[...], wrt_ref[...],
                     preferred_element_type=jnp.float32)
           + bl_ref[...])
    if relu:
        out = jnp.maximum(out, 0.0)
    out_ref[...] = out


def _tc_layer(p, cntp, h, wlt, wrt, bl, relu):
    grid = (N // RB,)
    return pl.pallas_call(
        functools.partial(_tc_body, relu),
        grid=grid,
        in_specs=[
            pl.BlockSpec((NC, RB, D), lambda i: (0, i, 0)),
            pl.BlockSpec((RB, NW), lambda i: (i, 0)),
            pl.BlockSpec((RB, D), lambda i: (i, 0)),
            pl.BlockSpec((D, D), lambda i: (0, 0)),
            pl.BlockSpec((D, D), lambda i: (0, 0)),
            pl.BlockSpec((1, D), lambda i: (0, 0)),
        ],
        out_specs=pl.BlockSpec((RB, D), lambda i: (i, 0)),
        out_shape=jax.ShapeDtypeStruct((N, D), jnp.float32),
    )(p, cntp, h, wlt, wrt, bl)


def kernel(x, edge_index, Wl0, bl0, Wr0, Wl1, bl1, Wr1, Wl2, bl2, Wr2):
    src = edge_index[0]
    dst = edge_index[1]
    pad = E_PAD - E
    pad_src = (jnp.arange(pad, dtype=jnp.int32) * 31) % N
    pad_dst = N + (jnp.arange(pad, dtype=jnp.int32) % (ACC_ROWS - N))
    src_p = jnp.concatenate([src, pad_src])
    dst_p = jnp.concatenate([dst, pad_dst])
    srcg = src_p.reshape(EPW, NW).T.reshape(NW, NCH, CHUNK)
    srcg = jnp.concatenate(
        [srcg, jnp.zeros((NW, 2, CHUNK), jnp.int32)], axis=1)
    dstg = dst_p.reshape(EPW, NW).T.reshape(NW, NCH, CHUNK)
    sc_agg_cnt, sc_agg = _sc_kernels()
    a0, cntp = sc_agg_cnt(x, srcg, dstg)
    cntp = cntp.reshape(NW, ACC_ROWS).T
    h1 = _tc_layer(a0, cntp, x, Wl0.T, Wr0.T, bl0.reshape(1, D), relu=True)
    a1 = sc_agg(h1, srcg, dstg)
    h2 = _tc_layer(a1, cntp, h1, Wl1.T, Wr1.T, bl1.reshape(1, D), relu=True)
    a2 = sc_agg(h2, srcg, dstg)
    h3 = _tc_layer(a2, cntp, h2, Wl2.T, Wr2.T, bl2.reshape(1, D), relu=False)
    return h3
